# baseline (speedup 1.0000x reference)
import functools

import jax
import jax.numpy as jnp
from jax.experimental import pallas as pl
from jax.experimental.pallas import tpu as pltpu


def _knn_body(rowsT_ref, colsT_ref, *out_refs, d, ks, kl, R, N, Npad):
    W = Npad // 8
    out_refs, dist_ref = out_refs[:-1], out_refs[-1]
    CW = min(256, W)
    nch = W // CW

    for c in range(nch):
        sl = slice(c * CW, (c + 1) * CW)

        def dd_body(dd, accs, sl=sl):
            cv = colsT_ref[dd, :, sl]
            return tuple(a + jnp.abs(rowsT_ref[0, dd, r] - cv)
                         for r, a in enumerate(accs))

        accs = jax.lax.fori_loop(
            0, d, dd_body,
            tuple(jnp.zeros((8, CW), jnp.float32) for _ in range(R)))
        for r in range(R):
            dist_ref[r, :, sl] = accs[r]

    lane = (jax.lax.broadcasted_iota(jnp.int32, (8, W), 0) * W
            + jax.lax.broadcasted_iota(jnp.int32, (8, W), 1))
    valid = lane < N
    row_io = jax.lax.broadcasted_iota(jnp.int32, (R, ks), 0)
    col_io = jax.lax.broadcasted_iota(jnp.int32, (R, ks), 1)
    Ms = jnp.zeros((R, ks), jnp.int32)
    if kl:
        row_io_l = jax.lax.broadcasted_iota(jnp.int32, (R, kl), 0)
        col_io_l = jax.lax.broadcasted_iota(jnp.int32, (R, kl), 1)
        Ml = jnp.zeros((R, kl), jnp.int32)
    for r in range(R):
        dr = dist_ref[r]
        ds = jnp.where(valid, dr, jnp.inf)
        for j in range(ks):
            m = jnp.min(ds)
            idx = jnp.min(jnp.where(ds <= m, lane, Npad))
            Ms = jnp.where((row_io == r) & (col_io == j), idx, Ms)
            ds = jnp.where(lane == idx, jnp.inf, ds)
        if kl:
            dl = jnp.where(valid, dr, -jnp.inf)
            for j in range(kl):
                m = jnp.max(dl)
                idx = jnp.min(jnp.where(dl >= m, lane, Npad))
                Ml = jnp.where((row_io_l == r) & (col_io_l == j), idx, Ml)
                dl = jnp.where(lane == idx, -jnp.inf, dl)
    out_refs[0][...] = Ms
    if kl:
        out_refs[1][...] = Ml


def _l1_knn_topk(xn, ks, kl):
    N, d = xn.shape
    R = 8
    C = 512
    Npad = ((N + C - 1) // C) * C
    xnT = jnp.pad(xn.T, ((0, 0), (0, Npad - N)))
    rows3 = xnT[:, :N].reshape(d, N // R, R).transpose(1, 0, 2)
    cols3 = xnT.reshape(d, 8, Npad // 8)

    outs = [jax.ShapeDtypeStruct((N, ks), jnp.int32)]
    out_specs = [pl.BlockSpec((R, ks), lambda i: (i, 0))]
    if kl:
        outs.append(jax.ShapeDtypeStruct((N, kl), jnp.int32))
        out_specs.append(pl.BlockSpec((R, kl), lambda i: (i, 0)))

    fn = pl.pallas_call(
        functools.partial(_knn_body, d=d, ks=ks, kl=kl, R=R, N=N, Npad=Npad),
        grid=(N // R,),
        in_specs=[
            pl.BlockSpec((1, d, R), lambda i: (i, 0, 0)),
            pl.BlockSpec((d, 8, Npad // 8), lambda i: (0, 0, 0)),
        ],
        out_specs=out_specs if kl else out_specs[0],
        out_shape=outs if kl else outs[0],
        scratch_shapes=[pltpu.VMEM((R, 8, Npad // 8), jnp.float32)],
    )
    return fn(rows3, cols3)


def _normalize(f):
    return f / jnp.linalg.norm(f, axis=1, keepdims=True)


def _edges_from(nbr_idx, k, n):
    src = jnp.repeat(jnp.arange(n), k)
    dst = nbr_idx.reshape(-1)
    mask = (src != dst).astype(jnp.float32)
    return src, dst, mask


def _gat_layer(x, src, dst, mask, W, a_s, a_d, b, n):
    h = x @ W
    alpha = (h * a_s)[src] + (h * a_d)[dst]
    alpha = jax.nn.leaky_relu(alpha, negative_slope=0.2)
    alpha = jnp.where(mask[:, None] > 0, alpha, -1e30)
    amax = jax.ops.segment_max(alpha, dst, num_segments=n)
    e = jnp.exp(alpha - amax[dst]) * mask[:, None]
    denom = jax.ops.segment_sum(e, dst, num_segments=n)
    coef = e / (denom[dst] + 1e-16)
    out = jax.ops.segment_sum(coef * h[src], dst, num_segments=n)
    return out.mean(axis=1, keepdims=True) + b


def _sage_layer(x, src, dst, mask, Wl, Wr, n):
    msg = x[src] * mask[:, None]
    s = jax.ops.segment_sum(msg, dst, num_segments=n)
    cnt = jax.ops.segment_sum(mask, dst, num_segments=n)
    mean = s / jnp.maximum(cnt, 1.0)[:, None]
    return mean @ Wl.T + x @ Wr.T


def kernel(x, fc1_W, fc1_b, fc2_W, fc2_b, fc3_W, fc3_b,
           gat1_W, gat1_as, gat1_ad, gat1_b,
           gat2_W, gat2_as, gat2_ad, gat2_b,
           gat3_W, gat3_as, gat3_ad, gat3_b,
           g1_Wl, g1_Wr, g2_Wl, g2_Wr, g3_Wl, g3_Wr,
           c21_Wl, c21_Wr,
           fc4_W, fc4_b, fc4n_W, fc4n_b, fc5_W, fc5_b,
           fc6_W, fc6_b, fc7_W, fc7_b):
    n = x.shape[0]
    K = 5
    x_price = x[:, 0:1]
    xf = x[:, 1:]
    x1 = jax.nn.relu(xf @ fc1_W.T + fc1_b)
    x2 = jax.nn.relu(x1 @ fc2_W.T + fc2_b)
    x3 = jax.nn.relu(x2 @ fc3_W.T + fc3_b)

    f1 = jnp.concatenate([x[:, 1:10], x[:, 12:18]], axis=1)
    f2 = jnp.concatenate([x[:, 1:3], x[:, 12:18]], axis=1)

    s1 = _l1_knn_topk(_normalize(f1), K, 0)
    s2 = _l1_knn_topk(_normalize(f2), K, 0)
    s3, l3 = _l1_knn_topk(_normalize(x3), 2 * K, 2 * K)

    src1, dst1, m1 = _edges_from(s1, K, n)
    src2, dst2, m2 = _edges_from(s2, K, n)
    src3, dst3, m3 = _edges_from(s3, 2 * K, n)
    nsrc = jnp.repeat(jnp.arange(n), 2 * K)
    ndst = l3.reshape(-1)
    nmask = jnp.ones_like(nsrc, dtype=jnp.float32)

    xp1 = jax.nn.relu(_gat_layer(x_price, src1, dst1, m1, gat1_W, gat1_as, gat1_ad, gat1_b, n))
    xp2 = jax.nn.relu(_gat_layer(x_price, src2, dst2, m2, gat2_W, gat2_as, gat2_ad, gat2_b, n))
    xp3 = jax.nn.relu(_gat_layer(x_price, src3, dst3, m3, gat3_W, gat3_as, gat3_ad, gat3_b, n))

    x1c = jnp.concatenate([x3, xp1, xp2, xp3], axis=1)
    x11 = jax.nn.relu(_sage_layer(x1c, src3, dst3, m3, g1_Wl, g1_Wr, n))
    x12 = jax.nn.relu(_sage_layer(x1c, src3, dst3, m3, g2_Wl, g2_Wr, n))
    x13 = jax.nn.relu(_sage_layer(x1c, src3, dst3, m3, g3_Wl, g3_Wr, n))
    x2c = jnp.concatenate([x11, x12, x13], axis=1)
    h = jax.nn.relu(x2c @ fc4_W.T + fc4_b)
    h = jax.nn.relu(h @ fc5_W.T + fc5_b)
    out = h @ fc6_W.T + fc6_b

    xn_ = jax.nn.relu(_sage_layer(x1c, nsrc, ndst, nmask, c21_Wl, c21_Wr, n))
    xn_ = jax.nn.relu(xn_ @ fc4n_W.T + fc4n_b)
    xn_ = xn_ @ fc7_W.T + fc7_b
    return out, xn_


# speedup vs baseline: 1.8323x; 1.8323x over previous
import functools

import jax
import jax.numpy as jnp
from jax.experimental import pallas as pl
from jax.experimental.pallas import tpu as pltpu


def _knn_body(rowsT_ref, colsT_ref, *out_refs, d, ks, kl, R, N, Npad):
    W = Npad // 8
    out_refs, dist_ref = out_refs[:-1], out_refs[-1]
    CW = min(256, W)
    nch = W // CW

    for c in range(nch):
        sl = slice(c * CW, (c + 1) * CW)

        def dd_body(dd, accs, sl=sl):
            cv = colsT_ref[dd, :, sl]
            return tuple(a + jnp.abs(rowsT_ref[0, dd, r] - cv)
                         for r, a in enumerate(accs))

        accs = jax.lax.fori_loop(
            0, d, dd_body,
            tuple(jnp.zeros((8, CW), jnp.float32) for _ in range(R)))
        for r in range(R):
            dist_ref[r, :, sl] = accs[r]

    lane = (jax.lax.broadcasted_iota(jnp.int32, (8, W), 0) * W
            + jax.lax.broadcasted_iota(jnp.int32, (8, W), 1))
    valid = lane < N
    row_io = jax.lax.broadcasted_iota(jnp.int32, (R, ks), 0)
    col_io = jax.lax.broadcasted_iota(jnp.int32, (R, ks), 1)
    Ms = jnp.zeros((R, ks), jnp.int32)
    if kl:
        row_io_l = jax.lax.broadcasted_iota(jnp.int32, (R, kl), 0)
        col_io_l = jax.lax.broadcasted_iota(jnp.int32, (R, kl), 1)
        Ml = jnp.zeros((R, kl), jnp.int32)
    for r in range(R):
        dr = dist_ref[r]
        ds = jnp.where(valid, dr, jnp.inf)
        for j in range(ks):
            m = jnp.min(ds)
            idx = jnp.min(jnp.where(ds <= m, lane, Npad))
            Ms = jnp.where((row_io == r) & (col_io == j), idx, Ms)
            ds = jnp.where(lane == idx, jnp.inf, ds)
        if kl:
            dl = jnp.where(valid, dr, -jnp.inf)
            for j in range(kl):
                m = jnp.max(dl)
                idx = jnp.min(jnp.where(dl >= m, lane, Npad))
                Ml = jnp.where((row_io_l == r) & (col_io_l == j), idx, Ml)
                dl = jnp.where(lane == idx, -jnp.inf, dl)
    out_refs[0][...] = Ms
    if kl:
        out_refs[1][...] = Ml


def _l1_knn_topk(xn, ks, kl):
    N, d = xn.shape
    R = 8
    C = 512
    Npad = ((N + C - 1) // C) * C
    xnT = jnp.pad(xn.T, ((0, 0), (0, Npad - N)))
    rows3 = xnT[:, :N].reshape(d, N // R, R).transpose(1, 0, 2)
    cols3 = xnT.reshape(d, 8, Npad // 8)

    outs = [jax.ShapeDtypeStruct((N, ks), jnp.int32)]
    out_specs = [pl.BlockSpec((R, ks), lambda i: (i, 0))]
    if kl:
        outs.append(jax.ShapeDtypeStruct((N, kl), jnp.int32))
        out_specs.append(pl.BlockSpec((R, kl), lambda i: (i, 0)))

    fn = pl.pallas_call(
        functools.partial(_knn_body, d=d, ks=ks, kl=kl, R=R, N=N, Npad=Npad),
        grid=(N // R,),
        in_specs=[
            pl.BlockSpec((1, d, R), lambda i: (i, 0, 0),
                         memory_space=pltpu.SMEM),
            pl.BlockSpec((d, 8, Npad // 8), lambda i: (0, 0, 0)),
        ],
        out_specs=out_specs if kl else out_specs[0],
        out_shape=outs if kl else outs[0],
        scratch_shapes=[pltpu.VMEM((R, 8, Npad // 8), jnp.float32)],
    )
    return fn(rows3, cols3)


def _normalize(f):
    return f / jnp.linalg.norm(f, axis=1, keepdims=True)


def _edges_from(nbr_idx, k, n):
    src = jnp.repeat(jnp.arange(n), k)
    dst = nbr_idx.reshape(-1)
    mask = (src != dst).astype(jnp.float32)
    return src, dst, mask


def _gat_layer(x, src, dst, mask, W, a_s, a_d, b, n):
    h = x @ W
    alpha = (h * a_s)[src] + (h * a_d)[dst]
    alpha = jax.nn.leaky_relu(alpha, negative_slope=0.2)
    alpha = jnp.where(mask[:, None] > 0, alpha, -1e30)
    amax = jax.ops.segment_max(alpha, dst, num_segments=n)
    e = jnp.exp(alpha - amax[dst]) * mask[:, None]
    denom = jax.ops.segment_sum(e, dst, num_segments=n)
    coef = e / (denom[dst] + 1e-16)
    out = jax.ops.segment_sum(coef * h[src], dst, num_segments=n)
    return out.mean(axis=1, keepdims=True) + b


def _sage_layer(x, src, dst, mask, Wl, Wr, n):
    msg = x[src] * mask[:, None]
    s = jax.ops.segment_sum(msg, dst, num_segments=n)
    cnt = jax.ops.segment_sum(mask, dst, num_segments=n)
    mean = s / jnp.maximum(cnt, 1.0)[:, None]
    return mean @ Wl.T + x @ Wr.T


def kernel(x, fc1_W, fc1_b, fc2_W, fc2_b, fc3_W, fc3_b,
           gat1_W, gat1_as, gat1_ad, gat1_b,
           gat2_W, gat2_as, gat2_ad, gat2_b,
           gat3_W, gat3_as, gat3_ad, gat3_b,
           g1_Wl, g1_Wr, g2_Wl, g2_Wr, g3_Wl, g3_Wr,
           c21_Wl, c21_Wr,
           fc4_W, fc4_b, fc4n_W, fc4n_b, fc5_W, fc5_b,
           fc6_W, fc6_b, fc7_W, fc7_b):
    n = x.shape[0]
    K = 5
    x_price = x[:, 0:1]
    xf = x[:, 1:]
    x1 = jax.nn.relu(xf @ fc1_W.T + fc1_b)
    x2 = jax.nn.relu(x1 @ fc2_W.T + fc2_b)
    x3 = jax.nn.relu(x2 @ fc3_W.T + fc3_b)

    f1 = jnp.concatenate([x[:, 1:10], x[:, 12:18]], axis=1)
    f2 = jnp.concatenate([x[:, 1:3], x[:, 12:18]], axis=1)

    s1 = _l1_knn_topk(_normalize(f1), K, 0)
    s2 = _l1_knn_topk(_normalize(f2), K, 0)
    s3, l3 = _l1_knn_topk(_normalize(x3), 2 * K, 2 * K)

    src1, dst1, m1 = _edges_from(s1, K, n)
    src2, dst2, m2 = _edges_from(s2, K, n)
    src3, dst3, m3 = _edges_from(s3, 2 * K, n)
    nsrc = jnp.repeat(jnp.arange(n), 2 * K)
    ndst = l3.reshape(-1)
    nmask = jnp.ones_like(nsrc, dtype=jnp.float32)

    xp1 = jax.nn.relu(_gat_layer(x_price, src1, dst1, m1, gat1_W, gat1_as, gat1_ad, gat1_b, n))
    xp2 = jax.nn.relu(_gat_layer(x_price, src2, dst2, m2, gat2_W, gat2_as, gat2_ad, gat2_b, n))
    xp3 = jax.nn.relu(_gat_layer(x_price, src3, dst3, m3, gat3_W, gat3_as, gat3_ad, gat3_b, n))

    x1c = jnp.concatenate([x3, xp1, xp2, xp3], axis=1)
    x11 = jax.nn.relu(_sage_layer(x1c, src3, dst3, m3, g1_Wl, g1_Wr, n))
    x12 = jax.nn.relu(_sage_layer(x1c, src3, dst3, m3, g2_Wl, g2_Wr, n))
    x13 = jax.nn.relu(_sage_layer(x1c, src3, dst3, m3, g3_Wl, g3_Wr, n))
    x2c = jnp.concatenate([x11, x12, x13], axis=1)
    h = jax.nn.relu(x2c @ fc4_W.T + fc4_b)
    h = jax.nn.relu(h @ fc5_W.T + fc5_b)
    out = h @ fc6_W.T + fc6_b

    xn_ = jax.nn.relu(_sage_layer(x1c, nsrc, ndst, nmask, c21_Wl, c21_Wr, n))
    xn_ = jax.nn.relu(xn_ @ fc4n_W.T + fc4n_b)
    xn_ = xn_ @ fc7_W.T + fc7_b
    return out, xn_


# speedup vs baseline: 7.7037x; 4.2044x over previous
import functools

import jax
import jax.numpy as jnp
from jax.experimental import pallas as pl
from jax.experimental.pallas import tpu as pltpu


def _knn_body(rowsT_ref, colsT_ref, *out_refs, d, ks, kl, R, N, Npad):
    W = Npad // 8
    out_refs, dist_ref = out_refs[:-1], out_refs[-1]
    CW = min(256, W)
    nch = W // CW

    for c in range(nch):
        sl = slice(c * CW, (c + 1) * CW)

        def dd_body(dd, accs, sl=sl):
            cv = colsT_ref[dd, :, sl]
            return tuple(a + jnp.abs(rowsT_ref[0, dd, r] - cv)
                         for r, a in enumerate(accs))

        accs = jax.lax.fori_loop(
            0, d, dd_body,
            tuple(jnp.zeros((8, CW), jnp.float32) for _ in range(R)))
        for r in range(R):
            dist_ref[r, :, sl] = accs[r]

    lane = (jax.lax.broadcasted_iota(jnp.int32, (8, W), 0) * W
            + jax.lax.broadcasted_iota(jnp.int32, (8, W), 1))
    valid = lane < N
    row_io = jax.lax.broadcasted_iota(jnp.int32, (R, ks), 0)
    col_io = jax.lax.broadcasted_iota(jnp.int32, (R, ks), 1)
    Ms = jnp.zeros((R, ks), jnp.int32)
    if kl:
        row_io_l = jax.lax.broadcasted_iota(jnp.int32, (R, kl), 0)
        col_io_l = jax.lax.broadcasted_iota(jnp.int32, (R, kl), 1)
        Ml = jnp.zeros((R, kl), jnp.int32)
    def _vmin2(a):
        return jnp.min(jnp.min(a, axis=1, keepdims=True), axis=0, keepdims=True)

    def _vmax2(a):
        return jnp.max(jnp.max(a, axis=1, keepdims=True), axis=0, keepdims=True)

    for r in range(R):
        dr = dist_ref[r]
        ds = jnp.where(valid, dr, jnp.inf)
        for j in range(ks):
            m = _vmin2(ds)
            idx = _vmin2(jnp.where(ds <= m, lane, Npad))
            Ms = jnp.where((row_io == r) & (col_io == j), idx, Ms)
            ds = jnp.where(lane == idx, jnp.inf, ds)
        if kl:
            dl = jnp.where(valid, dr, -jnp.inf)
            for j in range(kl):
                m = _vmax2(dl)
                idx = _vmin2(jnp.where(dl >= m, lane, Npad))
                Ml = jnp.where((row_io_l == r) & (col_io_l == j), idx, Ml)
                dl = jnp.where(lane == idx, -jnp.inf, dl)
    out_refs[0][...] = Ms
    if kl:
        out_refs[1][...] = Ml


def _l1_knn_topk(xn, ks, kl):
    N, d = xn.shape
    R = 8
    C = 512
    Npad = ((N + C - 1) // C) * C
    xnT = jnp.pad(xn.T, ((0, 0), (0, Npad - N)))
    rows3 = xnT[:, :N].reshape(d, N // R, R).transpose(1, 0, 2)
    cols3 = xnT.reshape(d, 8, Npad // 8)

    outs = [jax.ShapeDtypeStruct((N, ks), jnp.int32)]
    out_specs = [pl.BlockSpec((R, ks), lambda i: (i, 0))]
    if kl:
        outs.append(jax.ShapeDtypeStruct((N, kl), jnp.int32))
        out_specs.append(pl.BlockSpec((R, kl), lambda i: (i, 0)))

    fn = pl.pallas_call(
        functools.partial(_knn_body, d=d, ks=ks, kl=kl, R=R, N=N, Npad=Npad),
        grid=(N // R,),
        in_specs=[
            pl.BlockSpec((1, d, R), lambda i: (i, 0, 0),
                         memory_space=pltpu.SMEM),
            pl.BlockSpec((d, 8, Npad // 8), lambda i: (0, 0, 0)),
        ],
        out_specs=out_specs if kl else out_specs[0],
        out_shape=outs if kl else outs[0],
        scratch_shapes=[pltpu.VMEM((R, 8, Npad // 8), jnp.float32)],
    )
    return fn(rows3, cols3)


def _normalize(f):
    return f / jnp.linalg.norm(f, axis=1, keepdims=True)


def _edges_from(nbr_idx, k, n):
    src = jnp.repeat(jnp.arange(n), k)
    dst = nbr_idx.reshape(-1)
    mask = (src != dst).astype(jnp.float32)
    return src, dst, mask


def _gat_layer(x, src, dst, mask, W, a_s, a_d, b, n):
    h = x @ W
    alpha = (h * a_s)[src] + (h * a_d)[dst]
    alpha = jax.nn.leaky_relu(alpha, negative_slope=0.2)
    alpha = jnp.where(mask[:, None] > 0, alpha, -1e30)
    amax = jax.ops.segment_max(alpha, dst, num_segments=n)
    e = jnp.exp(alpha - amax[dst]) * mask[:, None]
    denom = jax.ops.segment_sum(e, dst, num_segments=n)
    coef = e / (denom[dst] + 1e-16)
    out = jax.ops.segment_sum(coef * h[src], dst, num_segments=n)
    return out.mean(axis=1, keepdims=True) + b


def _sage_layer(x, src, dst, mask, Wl, Wr, n):
    msg = x[src] * mask[:, None]
    s = jax.ops.segment_sum(msg, dst, num_segments=n)
    cnt = jax.ops.segment_sum(mask, dst, num_segments=n)
    mean = s / jnp.maximum(cnt, 1.0)[:, None]
    return mean @ Wl.T + x @ Wr.T


def kernel(x, fc1_W, fc1_b, fc2_W, fc2_b, fc3_W, fc3_b,
           gat1_W, gat1_as, gat1_ad, gat1_b,
           gat2_W, gat2_as, gat2_ad, gat2_b,
           gat3_W, gat3_as, gat3_ad, gat3_b,
           g1_Wl, g1_Wr, g2_Wl, g2_Wr, g3_Wl, g3_Wr,
           c21_Wl, c21_Wr,
           fc4_W, fc4_b, fc4n_W, fc4n_b, fc5_W, fc5_b,
           fc6_W, fc6_b, fc7_W, fc7_b):
    n = x.shape[0]
    K = 5
    x_price = x[:, 0:1]
    xf = x[:, 1:]
    x1 = jax.nn.relu(xf @ fc1_W.T + fc1_b)
    x2 = jax.nn.relu(x1 @ fc2_W.T + fc2_b)
    x3 = jax.nn.relu(x2 @ fc3_W.T + fc3_b)

    f1 = jnp.concatenate([x[:, 1:10], x[:, 12:18]], axis=1)
    f2 = jnp.concatenate([x[:, 1:3], x[:, 12:18]], axis=1)

    s1 = _l1_knn_topk(_normalize(f1), K, 0)
    s2 = _l1_knn_topk(_normalize(f2), K, 0)
    s3, l3 = _l1_knn_topk(_normalize(x3), 2 * K, 2 * K)

    src1, dst1, m1 = _edges_from(s1, K, n)
    src2, dst2, m2 = _edges_from(s2, K, n)
    src3, dst3, m3 = _edges_from(s3, 2 * K, n)
    nsrc = jnp.repeat(jnp.arange(n), 2 * K)
    ndst = l3.reshape(-1)
    nmask = jnp.ones_like(nsrc, dtype=jnp.float32)

    xp1 = jax.nn.relu(_gat_layer(x_price, src1, dst1, m1, gat1_W, gat1_as, gat1_ad, gat1_b, n))
    xp2 = jax.nn.relu(_gat_layer(x_price, src2, dst2, m2, gat2_W, gat2_as, gat2_ad, gat2_b, n))
    xp3 = jax.nn.relu(_gat_layer(x_price, src3, dst3, m3, gat3_W, gat3_as, gat3_ad, gat3_b, n))

    x1c = jnp.concatenate([x3, xp1, xp2, xp3], axis=1)
    x11 = jax.nn.relu(_sage_layer(x1c, src3, dst3, m3, g1_Wl, g1_Wr, n))
    x12 = jax.nn.relu(_sage_layer(x1c, src3, dst3, m3, g2_Wl, g2_Wr, n))
    x13 = jax.nn.relu(_sage_layer(x1c, src3, dst3, m3, g3_Wl, g3_Wr, n))
    x2c = jnp.concatenate([x11, x12, x13], axis=1)
    h = jax.nn.relu(x2c @ fc4_W.T + fc4_b)
    h = jax.nn.relu(h @ fc5_W.T + fc5_b)
    out = h @ fc6_W.T + fc6_b

    xn_ = jax.nn.relu(_sage_layer(x1c, nsrc, ndst, nmask, c21_Wl, c21_Wr, n))
    xn_ = jax.nn.relu(xn_ @ fc4n_W.T + fc4n_b)
    xn_ = xn_ @ fc7_W.T + fc7_b
    return out, xn_


# speedup vs baseline: 8.5789x; 1.1136x over previous
import functools

import jax
import jax.numpy as jnp
from jax.experimental import pallas as pl
from jax.experimental.pallas import tpu as pltpu


def _knn_body(rowsT_ref, colsT_ref, *out_refs, d, ks, kl, R, N, Npad):
    W = Npad // 8
    out_refs, dist_ref = out_refs[:-1], out_refs[-1]
    CW = min(256, W)
    nch = W // CW

    for c in range(nch):
        sl = slice(c * CW, (c + 1) * CW)

        def dd_body(dd, accs, sl=sl):
            cv = colsT_ref[dd, :, sl]
            return tuple(a + jnp.abs(rowsT_ref[0, dd, r] - cv)
                         for r, a in enumerate(accs))

        accs = jax.lax.fori_loop(
            0, d, dd_body,
            tuple(jnp.zeros((8, CW), jnp.float32) for _ in range(R)))
        for r in range(R):
            dist_ref[r, :, sl] = accs[r]

    lane = (jax.lax.broadcasted_iota(jnp.int32, (8, W), 0) * W
            + jax.lax.broadcasted_iota(jnp.int32, (8, W), 1))
    valid = lane < N
    row_io = jax.lax.broadcasted_iota(jnp.int32, (R, ks), 0)
    col_io = jax.lax.broadcasted_iota(jnp.int32, (R, ks), 1)
    Ms = jnp.zeros((R, ks), jnp.int32)
    if kl:
        row_io_l = jax.lax.broadcasted_iota(jnp.int32, (R, kl), 0)
        col_io_l = jax.lax.broadcasted_iota(jnp.int32, (R, kl), 1)
        Ml = jnp.zeros((R, kl), jnp.int32)
    def _vmin2(a):
        return jnp.min(jnp.min(a, axis=1, keepdims=True), axis=0, keepdims=True)

    def _vmax2(a):
        return jnp.max(jnp.max(a, axis=1, keepdims=True), axis=0, keepdims=True)

    for r in range(R):
        dr = dist_ref[r]
        ds = jnp.where(valid, dr, jnp.inf)
        for j in range(ks):
            m = _vmin2(ds)
            idx = _vmin2(jnp.where(ds <= m, lane, Npad))
            Ms = jnp.where((row_io == r) & (col_io == j), idx, Ms)
            ds = jnp.where(lane == idx, jnp.inf, ds)
        if kl:
            dl = jnp.where(valid, dr, -jnp.inf)
            for j in range(kl):
                m = _vmax2(dl)
                idx = _vmin2(jnp.where(dl >= m, lane, Npad))
                Ml = jnp.where((row_io_l == r) & (col_io_l == j), idx, Ml)
                dl = jnp.where(lane == idx, -jnp.inf, dl)
    out_refs[0][...] = Ms
    if kl:
        out_refs[1][...] = Ml


def _l1_knn_topk(xn, ks, kl):
    N, d = xn.shape
    R = 8
    C = 512
    Npad = ((N + C - 1) // C) * C
    xnT = jnp.pad(xn.T, ((0, 0), (0, Npad - N)))
    rows3 = xnT[:, :N].reshape(d, N // R, R).transpose(1, 0, 2)
    cols3 = xnT.reshape(d, 8, Npad // 8)

    outs = [jax.ShapeDtypeStruct((N, ks), jnp.int32)]
    out_specs = [pl.BlockSpec((R, ks), lambda i: (i, 0))]
    if kl:
        outs.append(jax.ShapeDtypeStruct((N, kl), jnp.int32))
        out_specs.append(pl.BlockSpec((R, kl), lambda i: (i, 0)))

    fn = pl.pallas_call(
        functools.partial(_knn_body, d=d, ks=ks, kl=kl, R=R, N=N, Npad=Npad),
        grid=(N // R,),
        in_specs=[
            pl.BlockSpec((1, d, R), lambda i: (i, 0, 0),
                         memory_space=pltpu.SMEM),
            pl.BlockSpec((d, 8, Npad // 8), lambda i: (0, 0, 0)),
        ],
        out_specs=out_specs if kl else out_specs[0],
        out_shape=outs if kl else outs[0],
        scratch_shapes=[pltpu.VMEM((R, 8, Npad // 8), jnp.float32)],
    )
    return fn(rows3, cols3)


def _normalize(f):
    return f / jnp.linalg.norm(f, axis=1, keepdims=True)


def _gat_scatter_body(p_ref, nbr_ref, uv_ref, s0_ref, s1_ref, *, K, Rb):
    pid = pl.program_id(0)

    @pl.when(pid == 0)
    def _():
        s0_ref[...] = jnp.zeros_like(s0_ref)
        s1_ref[...] = jnp.zeros_like(s1_ref)

    u = uv_ref[0:1, :]
    v = uv_ref[1:2, :]

    def body(il, carry):
        ig = pid * Rb + il
        ps = p_ref[ig]
        for k in range(K):
            j = nbr_ref[ig * K + k]
            pd = p_ref[j]
            msk = jnp.where(j == ig, 0.0, 1.0)
            z = ps * u + pd * v
            a = jnp.where(z > 0, z, 0.2 * z)
            w = jnp.exp(a) * msk
            s0_ref[pl.ds(j, 1), :] = s0_ref[pl.ds(j, 1), :] + w
            s1_ref[pl.ds(j, 1), :] = s1_ref[pl.ds(j, 1), :] + ps * w
        return carry

    jax.lax.fori_loop(0, Rb, body, 0)


def _gat_conv(p, nbr, W, a_s, a_d, b):
    N, K = nbr.shape
    H = W.shape[1]
    Rb = 1000 if N % 1000 == 0 else N
    uv = jnp.stack([W[0] * a_s, W[0] * a_d])
    fn = pl.pallas_call(
        functools.partial(_gat_scatter_body, K=K, Rb=Rb),
        grid=(N // Rb,),
        in_specs=[
            pl.BlockSpec(memory_space=pltpu.SMEM),
            pl.BlockSpec(memory_space=pltpu.SMEM),
            pl.BlockSpec((2, H), lambda i: (0, 0)),
        ],
        out_specs=[pl.BlockSpec((N, H), lambda i: (0, 0)),
                   pl.BlockSpec((N, H), lambda i: (0, 0))],
        out_shape=[jax.ShapeDtypeStruct((N, H), jnp.float32),
                   jax.ShapeDtypeStruct((N, H), jnp.float32)],
    )
    S0, S1 = fn(p, nbr.reshape(-1), uv)
    ratio = S1 / (S0 + 1e-16)
    return ratio @ (W[0] / H)[:, None] + b


def _sage_scatter_body(xw_ref, nbr_ref, ss_ref, *, K, Rb, self_mask):
    pid = pl.program_id(0)

    @pl.when(pid == 0)
    def _():
        ss_ref[...] = jnp.zeros_like(ss_ref)

    def body(il, carry):
        ig = pid * Rb + il
        xrow = xw_ref[pl.ds(il, 1), :]
        for k in range(K):
            j = nbr_ref[ig * K + k]
            if self_mask:
                msk = jnp.where(j == ig, 0.0, 1.0)
                upd = msk * xrow
            else:
                upd = xrow
            ss_ref[pl.ds(j, 1), :] = ss_ref[pl.ds(j, 1), :] + upd
        return carry

    jax.lax.fori_loop(0, Rb, body, 0)


def _sage_mean(x, nbr, self_mask):
    N, F = x.shape
    K = nbr.shape[1]
    Rb = 1000 if N % 1000 == 0 else N
    xw = jnp.concatenate([x, jnp.ones((N, 1), jnp.float32)], axis=1)
    fn = pl.pallas_call(
        functools.partial(_sage_scatter_body, K=K, Rb=Rb, self_mask=self_mask),
        grid=(N // Rb,),
        in_specs=[
            pl.BlockSpec((Rb, F + 1), lambda i: (i, 0)),
            pl.BlockSpec(memory_space=pltpu.SMEM),
        ],
        out_specs=pl.BlockSpec((N, F + 1), lambda i: (0, 0)),
        out_shape=jax.ShapeDtypeStruct((N, F + 1), jnp.float32),
    )
    ss = fn(xw, nbr.reshape(-1))
    return ss[:, :F] / jnp.maximum(ss[:, F:], 1.0)


def kernel(x, fc1_W, fc1_b, fc2_W, fc2_b, fc3_W, fc3_b,
           gat1_W, gat1_as, gat1_ad, gat1_b,
           gat2_W, gat2_as, gat2_ad, gat2_b,
           gat3_W, gat3_as, gat3_ad, gat3_b,
           g1_Wl, g1_Wr, g2_Wl, g2_Wr, g3_Wl, g3_Wr,
           c21_Wl, c21_Wr,
           fc4_W, fc4_b, fc4n_W, fc4n_b, fc5_W, fc5_b,
           fc6_W, fc6_b, fc7_W, fc7_b):
    n = x.shape[0]
    K = 5
    x_price = x[:, 0:1]
    xf = x[:, 1:]
    x1 = jax.nn.relu(xf @ fc1_W.T + fc1_b)
    x2 = jax.nn.relu(x1 @ fc2_W.T + fc2_b)
    x3 = jax.nn.relu(x2 @ fc3_W.T + fc3_b)

    f1 = jnp.concatenate([x[:, 1:10], x[:, 12:18]], axis=1)
    f2 = jnp.concatenate([x[:, 1:3], x[:, 12:18]], axis=1)

    s1 = _l1_knn_topk(_normalize(f1), K, 0)
    s2 = _l1_knn_topk(_normalize(f2), K, 0)
    s3, l3 = _l1_knn_topk(_normalize(x3), 2 * K, 2 * K)

    p = x[:, 0]
    xp1 = jax.nn.relu(_gat_conv(p, s1, gat1_W, gat1_as, gat1_ad, gat1_b))
    xp2 = jax.nn.relu(_gat_conv(p, s2, gat2_W, gat2_as, gat2_ad, gat2_b))
    xp3 = jax.nn.relu(_gat_conv(p, s3, gat3_W, gat3_as, gat3_ad, gat3_b))

    x1c = jnp.concatenate([x3, xp1, xp2, xp3], axis=1)
    mean3 = _sage_mean(x1c, s3, True)
    x11 = jax.nn.relu(mean3 @ g1_Wl.T + x1c @ g1_Wr.T)
    x12 = jax.nn.relu(mean3 @ g2_Wl.T + x1c @ g2_Wr.T)
    x13 = jax.nn.relu(mean3 @ g3_Wl.T + x1c @ g3_Wr.T)
    x2c = jnp.concatenate([x11, x12, x13], axis=1)
    h = jax.nn.relu(x2c @ fc4_W.T + fc4_b)
    h = jax.nn.relu(h @ fc5_W.T + fc5_b)
    out = h @ fc6_W.T + fc6_b

    meann = _sage_mean(x1c, l3, False)
    xn_ = jax.nn.relu(meann @ c21_Wl.T + x1c @ c21_Wr.T)
    xn_ = jax.nn.relu(xn_ @ fc4n_W.T + fc4n_b)
    xn_ = xn_ @ fc7_W.T + fc7_b
    return out, xn_


# speedup vs baseline: 10.0192x; 1.1679x over previous
import functools

import jax
import jax.numpy as jnp
from jax.experimental import pallas as pl
from jax.experimental.pallas import tpu as pltpu


def _knn_body(rowsT_ref, colsT_ref, *out_refs, d, ks, kl, R, N, Npad):
    W = Npad // 8
    out_refs, dist_ref = out_refs[:-1], out_refs[-1]
    CW = min(256, W)
    nch = W // CW

    for c in range(nch):
        sl = slice(c * CW, (c + 1) * CW)

        def dd_body(dd, accs, sl=sl):
            cv = colsT_ref[dd, :, sl]
            return tuple(a + jnp.abs(rowsT_ref[0, dd, r] - cv)
                         for r, a in enumerate(accs))

        accs = jax.lax.fori_loop(
            0, d, dd_body,
            tuple(jnp.zeros((8, CW), jnp.float32) for _ in range(R)))
        for r in range(R):
            dist_ref[r, :, sl] = accs[r]

    lane = (jax.lax.broadcasted_iota(jnp.int32, (8, W), 0) * W
            + jax.lax.broadcasted_iota(jnp.int32, (8, W), 1))
    valid = lane < N
    row_io = jax.lax.broadcasted_iota(jnp.int32, (R, ks), 0)
    col_io = jax.lax.broadcasted_iota(jnp.int32, (R, ks), 1)
    Ms = jnp.zeros((R, ks), jnp.int32)
    if kl:
        row_io_l = jax.lax.broadcasted_iota(jnp.int32, (R, kl), 0)
        col_io_l = jax.lax.broadcasted_iota(jnp.int32, (R, kl), 1)
        Ml = jnp.zeros((R, kl), jnp.int32)
    def _vmin2(a):
        return jnp.min(jnp.min(a, axis=1, keepdims=True), axis=0, keepdims=True)

    def _vmax2(a):
        return jnp.max(jnp.max(a, axis=1, keepdims=True), axis=0, keepdims=True)

    for r in range(R):
        dr = dist_ref[r]
        ds = jnp.where(valid, dr, jnp.inf)
        for j in range(ks):
            m = _vmin2(ds)
            idx = _vmin2(jnp.where(ds <= m, lane, Npad))
            Ms = jnp.where((row_io == r) & (col_io == j), idx, Ms)
            ds = jnp.where(lane == idx, jnp.inf, ds)
        if kl:
            dl = jnp.where(valid, dr, -jnp.inf)
            for j in range(kl):
                m = _vmax2(dl)
                idx = _vmin2(jnp.where(dl >= m, lane, Npad))
                Ml = jnp.where((row_io_l == r) & (col_io_l == j), idx, Ml)
                dl = jnp.where(lane == idx, -jnp.inf, dl)
    out_refs[0][...] = Ms
    if kl:
        out_refs[1][...] = Ml


def _l1_knn_topk(xn, ks, kl):
    N, d = xn.shape
    R = 16
    C = 512
    Npad = ((N + C - 1) // C) * C
    xnT = jnp.pad(xn.T, ((0, 0), (0, Npad - N)))
    rows3 = xnT[:, :N].reshape(d, N // R, R).transpose(1, 0, 2)
    cols3 = xnT.reshape(d, 8, Npad // 8)

    outs = [jax.ShapeDtypeStruct((N, ks), jnp.int32)]
    out_specs = [pl.BlockSpec((R, ks), lambda i: (i, 0))]
    if kl:
        outs.append(jax.ShapeDtypeStruct((N, kl), jnp.int32))
        out_specs.append(pl.BlockSpec((R, kl), lambda i: (i, 0)))

    fn = pl.pallas_call(
        functools.partial(_knn_body, d=d, ks=ks, kl=kl, R=R, N=N, Npad=Npad),
        grid=(N // R,),
        in_specs=[
            pl.BlockSpec((1, d, R), lambda i: (i, 0, 0),
                         memory_space=pltpu.SMEM),
            pl.BlockSpec((d, 8, Npad // 8), lambda i: (0, 0, 0)),
        ],
        out_specs=out_specs if kl else out_specs[0],
        out_shape=outs if kl else outs[0],
        scratch_shapes=[pltpu.VMEM((R, 8, Npad // 8), jnp.float32)],
    )
    return fn(rows3, cols3)


def _normalize(f):
    return f / jnp.linalg.norm(f, axis=1, keepdims=True)


def _gat_scatter_body(p_ref, nbr_ref, uv_ref, s0_ref, s1_ref, *, K, Rb):
    pid = pl.program_id(0)

    @pl.when(pid == 0)
    def _():
        s0_ref[...] = jnp.zeros_like(s0_ref)
        s1_ref[...] = jnp.zeros_like(s1_ref)

    u = uv_ref[0:1, :]
    v = uv_ref[1:2, :]

    def body(il, carry):
        ig = pid * Rb + il
        ps = p_ref[ig]
        for k in range(K):
            j = nbr_ref[ig * K + k]
            pd = p_ref[j]
            msk = jnp.where(j == ig, 0.0, 1.0)
            z = ps * u + pd * v
            a = jnp.where(z > 0, z, 0.2 * z)
            w = jnp.exp(a) * msk
            s0_ref[pl.ds(j, 1), :] = s0_ref[pl.ds(j, 1), :] + w
            s1_ref[pl.ds(j, 1), :] = s1_ref[pl.ds(j, 1), :] + ps * w
        return carry

    jax.lax.fori_loop(0, Rb, body, 0)


def _gat_conv(p, nbr, W, a_s, a_d, b):
    N, K = nbr.shape
    H = W.shape[1]
    Rb = 1000 if N % 1000 == 0 else N
    uv = jnp.stack([W[0] * a_s, W[0] * a_d])
    fn = pl.pallas_call(
        functools.partial(_gat_scatter_body, K=K, Rb=Rb),
        grid=(N // Rb,),
        in_specs=[
            pl.BlockSpec(memory_space=pltpu.SMEM),
            pl.BlockSpec(memory_space=pltpu.SMEM),
            pl.BlockSpec((2, H), lambda i: (0, 0)),
        ],
        out_specs=[pl.BlockSpec((N, H), lambda i: (0, 0)),
                   pl.BlockSpec((N, H), lambda i: (0, 0))],
        out_shape=[jax.ShapeDtypeStruct((N, H), jnp.float32),
                   jax.ShapeDtypeStruct((N, H), jnp.float32)],
    )
    S0, S1 = fn(p, nbr.reshape(-1), uv)
    ratio = S1 / (S0 + 1e-16)
    return ratio @ (W[0] / H)[:, None] + b


def _sage_scatter_body(xw_ref, nbr_ref, ss_ref, *, K, Rb, self_mask):
    pid = pl.program_id(0)

    @pl.when(pid == 0)
    def _():
        ss_ref[...] = jnp.zeros_like(ss_ref)

    def body(il, carry):
        ig = pid * Rb + il
        xrow = xw_ref[pl.ds(il, 1), :]
        for k in range(K):
            j = nbr_ref[ig * K + k]
            if self_mask:
                msk = jnp.where(j == ig, 0.0, 1.0)
                upd = msk * xrow
            else:
                upd = xrow
            ss_ref[pl.ds(j, 1), :] = ss_ref[pl.ds(j, 1), :] + upd
        return carry

    jax.lax.fori_loop(0, Rb, body, 0)


def _sage_mean(x, nbr, self_mask):
    N, F = x.shape
    K = nbr.shape[1]
    Rb = 1000 if N % 1000 == 0 else N
    xw = jnp.concatenate([x, jnp.ones((N, 1), jnp.float32)], axis=1)
    fn = pl.pallas_call(
        functools.partial(_sage_scatter_body, K=K, Rb=Rb, self_mask=self_mask),
        grid=(N // Rb,),
        in_specs=[
            pl.BlockSpec((Rb, F + 1), lambda i: (i, 0)),
            pl.BlockSpec(memory_space=pltpu.SMEM),
        ],
        out_specs=pl.BlockSpec((N, F + 1), lambda i: (0, 0)),
        out_shape=jax.ShapeDtypeStruct((N, F + 1), jnp.float32),
    )
    ss = fn(xw, nbr.reshape(-1))
    return ss[:, :F] / jnp.maximum(ss[:, F:], 1.0)


def kernel(x, fc1_W, fc1_b, fc2_W, fc2_b, fc3_W, fc3_b,
           gat1_W, gat1_as, gat1_ad, gat1_b,
           gat2_W, gat2_as, gat2_ad, gat2_b,
           gat3_W, gat3_as, gat3_ad, gat3_b,
           g1_Wl, g1_Wr, g2_Wl, g2_Wr, g3_Wl, g3_Wr,
           c21_Wl, c21_Wr,
           fc4_W, fc4_b, fc4n_W, fc4n_b, fc5_W, fc5_b,
           fc6_W, fc6_b, fc7_W, fc7_b):
    n = x.shape[0]
    K = 5
    x_price = x[:, 0:1]
    xf = x[:, 1:]
    x1 = jax.nn.relu(xf @ fc1_W.T + fc1_b)
    x2 = jax.nn.relu(x1 @ fc2_W.T + fc2_b)
    x3 = jax.nn.relu(x2 @ fc3_W.T + fc3_b)

    f1 = jnp.concatenate([x[:, 1:10], x[:, 12:18]], axis=1)
    f2 = jnp.concatenate([x[:, 1:3], x[:, 12:18]], axis=1)

    s1 = _l1_knn_topk(_normalize(f1), K, 0)
    s2 = _l1_knn_topk(_normalize(f2), K, 0)
    s3, l3 = _l1_knn_topk(_normalize(x3), 2 * K, 2 * K)

    p = x[:, 0]
    xp1 = jax.nn.relu(_gat_conv(p, s1, gat1_W, gat1_as, gat1_ad, gat1_b))
    xp2 = jax.nn.relu(_gat_conv(p, s2, gat2_W, gat2_as, gat2_ad, gat2_b))
    xp3 = jax.nn.relu(_gat_conv(p, s3, gat3_W, gat3_as, gat3_ad, gat3_b))

    x1c = jnp.concatenate([x3, xp1, xp2, xp3], axis=1)
    mean3 = _sage_mean(x1c, s3, True)
    x11 = jax.nn.relu(mean3 @ g1_Wl.T + x1c @ g1_Wr.T)
    x12 = jax.nn.relu(mean3 @ g2_Wl.T + x1c @ g2_Wr.T)
    x13 = jax.nn.relu(mean3 @ g3_Wl.T + x1c @ g3_Wr.T)
    x2c = jnp.concatenate([x11, x12, x13], axis=1)
    h = jax.nn.relu(x2c @ fc4_W.T + fc4_b)
    h = jax.nn.relu(h @ fc5_W.T + fc5_b)
    out = h @ fc6_W.T + fc6_b

    meann = _sage_mean(x1c, l3, False)
    xn_ = jax.nn.relu(meann @ c21_Wl.T + x1c @ c21_Wr.T)
    xn_ = jax.nn.relu(xn_ @ fc4n_W.T + fc4n_b)
    xn_ = xn_ @ fc7_W.T + fc7_b
    return out, xn_
